# baseline (speedup 1.0000x reference)
import functools

import jax
import jax.numpy as jnp
import numpy as np
from jax import lax
from jax.experimental import pallas as pl
from jax.experimental.pallas import tpu as pltpu
from jax.experimental.pallas import tpu_sc as plsc

_TOPK_RATIO = 0.1
_MININT = np.int32(-(2 ** 31))
_LANES = 16


def _fused_mm_kernel(x_ref, w1_ref, b1_ref, wa1_ref, ba1_ref, w2_ref, b2_ref,
                     wa2_ref, ba2_ref, seg_ref, attn_ref):
    x = x_ref[...]
    t1 = jax.lax.dot_general(x, w1_ref[...], (((1,), (1,)), ((), ())),
                             preferred_element_type=jnp.float32)
    h = jnp.maximum(t1 + b1_ref[...], 0.0)
    seg = jax.lax.dot_general(h, w2_ref[...], (((1,), (1,)), ((), ())),
                              preferred_element_type=jnp.float32) + b2_ref[...]
    seg_ref[...] = seg
    t2 = jax.lax.dot_general(x, wa1_ref[...], (((1,), (1,)), ((), ())),
                             preferred_element_type=jnp.float32)
    ha = jnp.tanh(t2 + ba1_ref[...])
    a = jax.lax.dot_general(ha, wa2_ref[...], (((1,), (1,)), ((), ())),
                            preferred_element_type=jnp.float32)
    attn_ref[...] = a[:, 0:1] + ba2_ref[0, 0]


def _cumsum_lanes(v):
    B, T = v.shape
    s = 1
    while s < T:
        v = v + jnp.concatenate(
            [jnp.zeros((B, s), v.dtype), v[:, :T - s]], axis=1)
        s *= 2
    return v


def _weights_kernel(attn_ref, w_ref, idx_ref, *, k, kpad):
    a = attn_ref[...]
    B, T = a.shape
    bits = jax.lax.bitcast_convert_type(a, jnp.int32)
    sk = jnp.where(bits < 0,
                   jnp.bitwise_xor(jnp.bitwise_not(bits), _MININT),
                   bits)

    def body(i, p):
        b = jnp.int32(31) - i
        cand = jnp.bitwise_or(p, jnp.left_shift(jnp.int32(1), b))
        icand = jnp.bitwise_xor(cand, _MININT)
        cnt = jnp.sum((sk >= icand).astype(jnp.int32), axis=1, keepdims=True)
        return jnp.where(cnt >= k, cand, p)

    p = jax.lax.fori_loop(0, 32, body, jnp.zeros((B, 1), jnp.int32))
    ithr = jnp.bitwise_xor(p, _MININT)

    gt = sk > ithr
    c_gt = jnp.sum(gt.astype(jnp.int32), axis=1, keepdims=True)
    eq = sk == ithr
    r = jnp.int32(k) - c_gt
    e = _cumsum_lanes(eq.astype(jnp.int32))
    sel = jnp.logical_or(gt, jnp.logical_and(eq, e <= r))
    mask = jnp.where(sel, jnp.float32(1.0 / k), jnp.float32(0.0))
    ssum = jnp.sum(mask, axis=1, keepdims=True)
    w_ref[...] = mask / (ssum + jnp.float32(1e-8))

    pos = _cumsum_lanes(sel.astype(jnp.int32)) - 1
    pos = jnp.where(sel, pos, jnp.int32(-1))
    t_iota = jax.lax.broadcasted_iota(jnp.int32, (1, T), 1)
    r_iota = jax.lax.broadcasted_iota(jnp.int32, (kpad, 1), 0)
    rows = []
    for b in range(B):
        onehot = (pos[b:b + 1] == r_iota)
        contrib = jnp.sum(jnp.where(onehot, t_iota, 0), axis=1)
        rows.append(contrib + jnp.int32(b * T))
    idx_ref[...] = jnp.stack(rows, axis=0)


def _make_sc_pool(B, T, C, k, kpad, ch):
    nch = kpad // ch
    csl = C // _LANES
    w0 = np.float32(np.float32(1.0 / k) /
                    (np.float32(k * np.float64(np.float32(1.0 / k)))
                     + np.float32(1e-8)))
    mesh = plsc.VectorSubcoreMesh(core_axis_name="c", subcore_axis_name="s")

    @functools.partial(
        pl.kernel, mesh=mesh,
        out_type=jax.ShapeDtypeStruct((B, C), jnp.float32),
        scratch_types=(
            [pltpu.VMEM((ch,), jnp.int32) for _ in range(nch)]
            + [pltpu.VMEM((kpad, C), jnp.float32),
               pltpu.VMEM((C,), jnp.float32),
               pltpu.SemaphoreType.DMA]
        ),
    )
    def sc_pool(idx_hbm, seg_hbm, clip_hbm, *refs):
        idxv = refs[:nch]
        rows, clipv, sem = refs[nch], refs[nch + 1], refs[nch + 2]
        wid = lax.axis_index("s") * 2 + lax.axis_index("c")

        @pl.when(wid < B)
        def _():
            base = wid * kpad
            for c in range(nch):
                pltpu.sync_copy(idx_hbm.at[pl.ds(base + c * ch, ch)], idxv[c])
            copies = [
                pltpu.async_copy(seg_hbm.at[idxv[c]],
                                 rows.at[pl.ds(c * ch, ch)], sem)
                for c in range(nch)
            ]
            for cp in copies:
                cp.wait()

            def acc_body(j, accs):
                return tuple(accs[s] + rows[j, pl.ds(s * _LANES, _LANES)]
                             for s in range(csl))
            accs = tuple(jnp.zeros((_LANES,), jnp.float32)
                         for _ in range(csl))
            accs = lax.fori_loop(0, k, acc_body, accs)
            for s in range(csl):
                clipv[pl.ds(s * _LANES, _LANES)] = accs[s] * w0
            pltpu.sync_copy(clipv, clip_hbm.at[wid])

    return sc_pool


def kernel(x, W1, b1, W2, b2, Wa1, ba1, Wa2, ba2):
    B, T, D = x.shape
    HID = W1.shape[0]
    C = W2.shape[0]
    k = max(1, min(T, int(round(T * _TOPK_RATIO))))
    kpad = ((k + _LANES - 1) // _LANES) * _LANES
    ch = None
    for cand in range(min(128, kpad), 7, -1):
        if cand % 8 == 0 and kpad % cand == 0:
            ch = cand
            break
    M = B * T
    TM = 256 if M % 256 == 0 else T

    xf = x.reshape(M, D)
    b1r = b1.reshape(1, HID)
    ba1r = ba1.reshape(1, HID)
    b2r = b2.reshape(1, C)
    ba2r = ba2.reshape(1, 1)
    wa2p = jnp.zeros((128, HID), jnp.float32).at[0].set(Wa2[0])

    seg_flat, attn_flat = pl.pallas_call(
        _fused_mm_kernel,
        grid=(M // TM,),
        in_specs=[
            pl.BlockSpec((TM, D), lambda i: (i, 0)),
            pl.BlockSpec((HID, D), lambda i: (0, 0)),
            pl.BlockSpec((1, HID), lambda i: (0, 0)),
            pl.BlockSpec((HID, D), lambda i: (0, 0)),
            pl.BlockSpec((1, HID), lambda i: (0, 0)),
            pl.BlockSpec((C, HID), lambda i: (0, 0)),
            pl.BlockSpec((1, C), lambda i: (0, 0)),
            pl.BlockSpec((128, HID), lambda i: (0, 0)),
            pl.BlockSpec((1, 1), lambda i: (0, 0)),
        ],
        out_specs=[
            pl.BlockSpec((TM, C), lambda i: (i, 0)),
            pl.BlockSpec((TM, 1), lambda i: (i, 0)),
        ],
        out_shape=[
            jax.ShapeDtypeStruct((M, C), jnp.float32),
            jax.ShapeDtypeStruct((M, 1), jnp.float32),
        ],
    )(xf, W1, b1r, Wa1, ba1r, W2, b2r, wa2p, ba2r)

    seg_logits = seg_flat.reshape(B, T, C)
    attn = attn_flat.reshape(B, T)

    weights, idx = pl.pallas_call(
        functools.partial(_weights_kernel, k=k, kpad=kpad),
        grid=(1,),
        in_specs=[pl.BlockSpec((B, T), lambda i: (0, 0))],
        out_specs=[
            pl.BlockSpec((B, T), lambda i: (0, 0)),
            pl.BlockSpec((B, kpad), lambda i: (0, 0)),
        ],
        out_shape=[
            jax.ShapeDtypeStruct((B, T), jnp.float32),
            jax.ShapeDtypeStruct((B, kpad), jnp.int32),
        ],
    )(attn)

    if ch is None:
        def _pool_kernel(w_ref, seg_ref, clip_ref):
            clip_ref[0] = jax.lax.dot_general(
                w_ref[0], seg_ref[0], (((1,), (0,)), ((), ())),
                preferred_element_type=jnp.float32)
        clip_logits = pl.pallas_call(
            _pool_kernel,
            grid=(B,),
            in_specs=[
                pl.BlockSpec((1, 1, T), lambda b: (b, 0, 0)),
                pl.BlockSpec((1, T, C), lambda b: (b, 0, 0)),
            ],
            out_specs=[pl.BlockSpec((1, 1, C), lambda b: (b, 0, 0))],
            out_shape=[jax.ShapeDtypeStruct((B, 1, C), jnp.float32)],
        )(weights.reshape(B, 1, T), seg_logits)[0].reshape(B, C)
    else:
        sc_pool = _make_sc_pool(B, T, C, k, kpad, ch)
        clip_logits = sc_pool(idx.reshape(B * kpad), seg_flat)

    return clip_logits, seg_logits, weights


# speedup vs baseline: 1.2933x; 1.2933x over previous
import functools

import jax
import jax.numpy as jnp
import numpy as np
from jax import lax
from jax.experimental import pallas as pl
from jax.experimental.pallas import tpu as pltpu
from jax.experimental.pallas import tpu_sc as plsc

_TOPK_RATIO = 0.1
_MININT = np.int32(-(2 ** 31))
_LANES = 16


def _fused_mm_kernel(x_ref, w1_ref, b1_ref, wa1_ref, ba1_ref, w2_ref, b2_ref,
                     wa2_ref, ba2_ref, seg_ref, attn_ref):
    x = x_ref[...]
    t1 = jax.lax.dot_general(x, w1_ref[...], (((1,), (1,)), ((), ())),
                             preferred_element_type=jnp.float32)
    h = jnp.maximum(t1 + b1_ref[...], 0.0)
    seg = jax.lax.dot_general(h, w2_ref[...], (((1,), (1,)), ((), ())),
                              preferred_element_type=jnp.float32) + b2_ref[...]
    seg_ref[...] = seg
    t2 = jax.lax.dot_general(x, wa1_ref[...], (((1,), (1,)), ((), ())),
                             preferred_element_type=jnp.float32)
    ha = jnp.tanh(t2 + ba1_ref[...])
    a = jax.lax.dot_general(ha, wa2_ref[...], (((1,), (1,)), ((), ())),
                            preferred_element_type=jnp.float32)
    attn_ref[...] = a[:, 0:1] + ba2_ref[0, 0]


def _cumsum_lanes(v):
    B, T = v.shape
    s = 1
    while s < T:
        v = v + jnp.concatenate(
            [jnp.zeros((B, s), v.dtype), v[:, :T - s]], axis=1)
        s *= 2
    return v


def _weights_kernel(attn_ref, w_ref, idx_ref, *, k, kpad):
    a = attn_ref[...]
    B, T = a.shape
    bits = jax.lax.bitcast_convert_type(a, jnp.int32)
    sk = jnp.where(bits < 0,
                   jnp.bitwise_xor(jnp.bitwise_not(bits), _MININT),
                   bits)

    def body(i, p):
        b = jnp.int32(31) - i
        cand = jnp.bitwise_or(p, jnp.left_shift(jnp.int32(1), b))
        icand = jnp.bitwise_xor(cand, _MININT)
        cnt = jnp.sum((sk >= icand).astype(jnp.int32), axis=1, keepdims=True)
        return jnp.where(cnt >= k, cand, p)

    p = jax.lax.fori_loop(0, 32, body, jnp.zeros((B, 1), jnp.int32))
    ithr = jnp.bitwise_xor(p, _MININT)

    gt = sk > ithr
    c_gt = jnp.sum(gt.astype(jnp.int32), axis=1, keepdims=True)
    eq = sk == ithr
    r = jnp.int32(k) - c_gt
    e = _cumsum_lanes(eq.astype(jnp.int32))
    sel = jnp.logical_or(gt, jnp.logical_and(eq, e <= r))
    mask = jnp.where(sel, jnp.float32(1.0 / k), jnp.float32(0.0))
    ssum = jnp.sum(mask, axis=1, keepdims=True)
    w_ref[...] = mask / (ssum + jnp.float32(1e-8))

    pos = _cumsum_lanes(sel.astype(jnp.int32)) - 1
    pos = jnp.where(sel, pos, jnp.int32(-1))
    t_iota = jax.lax.broadcasted_iota(jnp.int32, (1, T), 1)
    r_iota = jax.lax.broadcasted_iota(jnp.int32, (kpad, 1), 0)
    rows = []
    for b in range(B):
        onehot = (pos[b:b + 1] == r_iota)
        contrib = jnp.sum(jnp.where(onehot, t_iota, 0), axis=1)
        rows.append(contrib + jnp.int32(b * T))
    idx_ref[...] = jnp.stack(rows, axis=0)


def _make_sc_pool(B, T, C, k, kpad, ch):
    nch = kpad // ch
    csl = C // _LANES
    w0 = np.float32(np.float32(1.0 / k) /
                    (np.float32(k * np.float64(np.float32(1.0 / k)))
                     + np.float32(1e-8)))
    mesh = plsc.VectorSubcoreMesh(core_axis_name="c", subcore_axis_name="s")

    @functools.partial(
        pl.kernel, mesh=mesh,
        out_type=jax.ShapeDtypeStruct((B, C), jnp.float32),
        scratch_types=(
            [pltpu.VMEM((ch,), jnp.int32) for _ in range(nch)]
            + [pltpu.VMEM((kpad, C), jnp.float32),
               pltpu.VMEM((C,), jnp.float32),
               pltpu.SemaphoreType.DMA]
        ),
    )
    def sc_pool(idx_hbm, seg_hbm, clip_hbm, *refs):
        idxv = refs[:nch]
        rows, clipv, sem = refs[nch], refs[nch + 1], refs[nch + 2]
        wid = lax.axis_index("s") * 2 + lax.axis_index("c")

        @pl.when(wid < B)
        def _():
            base = wid * kpad
            for c in range(nch):
                pltpu.sync_copy(idx_hbm.at[pl.ds(base + c * ch, ch)], idxv[c])
            copies = [
                pltpu.async_copy(seg_hbm.at[idxv[c]],
                                 rows.at[pl.ds(c * ch, ch)], sem)
                for c in range(nch)
            ]
            for cp in copies:
                cp.wait()

            def acc_body(j, accs):
                return tuple(accs[s] + rows[j, pl.ds(s * _LANES, _LANES)]
                             for s in range(csl))
            accs = tuple(jnp.zeros((_LANES,), jnp.float32)
                         for _ in range(csl))
            accs = lax.fori_loop(0, k, acc_body, accs)
            for s in range(csl):
                clipv[pl.ds(s * _LANES, _LANES)] = accs[s] * w0
            pltpu.sync_copy(clipv, clip_hbm.at[wid])

    return sc_pool


def kernel(x, W1, b1, W2, b2, Wa1, ba1, Wa2, ba2):
    B, T, D = x.shape
    HID = W1.shape[0]
    C = W2.shape[0]
    k = max(1, min(T, int(round(T * _TOPK_RATIO))))
    kpad = ((k + _LANES - 1) // _LANES) * _LANES
    ch = None
    for cand in range(min(128, kpad), 7, -1):
        if cand % 8 == 0 and kpad % cand == 0:
            ch = cand
            break
    M = B * T
    TM = 1024 if M % 1024 == 0 else T

    xf = x.reshape(M, D)
    b1r = b1.reshape(1, HID)
    ba1r = ba1.reshape(1, HID)
    b2r = b2.reshape(1, C)
    ba2r = ba2.reshape(1, 1)
    wa2p = jnp.zeros((128, HID), jnp.float32).at[0].set(Wa2[0])

    seg_flat, attn_flat = pl.pallas_call(
        _fused_mm_kernel,
        grid=(M // TM,),
        in_specs=[
            pl.BlockSpec((TM, D), lambda i: (i, 0)),
            pl.BlockSpec((HID, D), lambda i: (0, 0)),
            pl.BlockSpec((1, HID), lambda i: (0, 0)),
            pl.BlockSpec((HID, D), lambda i: (0, 0)),
            pl.BlockSpec((1, HID), lambda i: (0, 0)),
            pl.BlockSpec((C, HID), lambda i: (0, 0)),
            pl.BlockSpec((1, C), lambda i: (0, 0)),
            pl.BlockSpec((128, HID), lambda i: (0, 0)),
            pl.BlockSpec((1, 1), lambda i: (0, 0)),
        ],
        out_specs=[
            pl.BlockSpec((TM, C), lambda i: (i, 0)),
            pl.BlockSpec((TM, 1), lambda i: (i, 0)),
        ],
        out_shape=[
            jax.ShapeDtypeStruct((M, C), jnp.float32),
            jax.ShapeDtypeStruct((M, 1), jnp.float32),
        ],
    )(xf, W1, b1r, Wa1, ba1r, W2, b2r, wa2p, ba2r)

    seg_logits = seg_flat.reshape(B, T, C)
    attn = attn_flat.reshape(B, T)

    weights, idx = pl.pallas_call(
        functools.partial(_weights_kernel, k=k, kpad=kpad),
        grid=(1,),
        in_specs=[pl.BlockSpec((B, T), lambda i: (0, 0))],
        out_specs=[
            pl.BlockSpec((B, T), lambda i: (0, 0)),
            pl.BlockSpec((B, kpad), lambda i: (0, 0)),
        ],
        out_shape=[
            jax.ShapeDtypeStruct((B, T), jnp.float32),
            jax.ShapeDtypeStruct((B, kpad), jnp.int32),
        ],
    )(attn)

    if ch is None:
        def _pool_kernel(w_ref, seg_ref, clip_ref):
            clip_ref[0] = jax.lax.dot_general(
                w_ref[0], seg_ref[0], (((1,), (0,)), ((), ())),
                preferred_element_type=jnp.float32)
        clip_logits = pl.pallas_call(
            _pool_kernel,
            grid=(B,),
            in_specs=[
                pl.BlockSpec((1, 1, T), lambda b: (b, 0, 0)),
                pl.BlockSpec((1, T, C), lambda b: (b, 0, 0)),
            ],
            out_specs=[pl.BlockSpec((1, 1, C), lambda b: (b, 0, 0))],
            out_shape=[jax.ShapeDtypeStruct((B, 1, C), jnp.float32)],
        )(weights.reshape(B, 1, T), seg_logits)[0].reshape(B, C)
    else:
        sc_pool = _make_sc_pool(B, T, C, k, kpad, ch)
        clip_logits = sc_pool(idx.reshape(B * kpad), seg_flat)

    return clip_logits, seg_logits, weights


# speedup vs baseline: 1.3216x; 1.0219x over previous
import functools

import jax
import jax.numpy as jnp
import numpy as np
from jax import lax
from jax.experimental import pallas as pl
from jax.experimental.pallas import tpu as pltpu
from jax.experimental.pallas import tpu_sc as plsc

_TOPK_RATIO = 0.1
_MININT = np.int32(-(2 ** 31))
_LANES = 16


def _fused_mm_kernel(x_ref, w1_ref, b1_ref, wa1_ref, ba1_ref, w2_ref, b2_ref,
                     wa2_ref, ba2_ref, seg_ref, attn_ref):
    x = x_ref[...]
    t1 = jax.lax.dot_general(x, w1_ref[...], (((1,), (1,)), ((), ())),
                             preferred_element_type=jnp.float32)
    h = jnp.maximum(t1 + b1_ref[...], 0.0)
    seg = jax.lax.dot_general(h, w2_ref[...], (((1,), (1,)), ((), ())),
                              preferred_element_type=jnp.float32) + b2_ref[...]
    seg_ref[...] = seg
    t2 = jax.lax.dot_general(x, wa1_ref[...], (((1,), (1,)), ((), ())),
                             preferred_element_type=jnp.float32)
    ha = jnp.tanh(t2 + ba1_ref[...])
    a = jax.lax.dot_general(ha, wa2_ref[...], (((1,), (1,)), ((), ())),
                            preferred_element_type=jnp.float32)
    attn_ref[...] = a[:, 0:1] + ba2_ref[0, 0]


def _cumsum_lanes(v):
    B, T = v.shape
    s = 1
    while s < T:
        v = v + jnp.concatenate(
            [jnp.zeros((B, s), v.dtype), v[:, :T - s]], axis=1)
        s *= 2
    return v


def _weights_kernel(attn_ref, w_ref, idx_ref, *, k, kpad):
    a = attn_ref[...]
    B, T = a.shape
    bits = jax.lax.bitcast_convert_type(a, jnp.int32)
    sk = jnp.where(bits < 0,
                   jnp.bitwise_xor(jnp.bitwise_not(bits), _MININT),
                   bits)

    def body(i, p):
        b = jnp.int32(31) - i
        cand = jnp.bitwise_or(p, jnp.left_shift(jnp.int32(1), b))
        icand = jnp.bitwise_xor(cand, _MININT)
        cnt = jnp.sum((sk >= icand).astype(jnp.int32), axis=1, keepdims=True)
        return jnp.where(cnt >= k, cand, p)

    p = jax.lax.fori_loop(0, 32, body, jnp.zeros((B, 1), jnp.int32))
    ithr = jnp.bitwise_xor(p, _MININT)

    gt = sk > ithr
    c_gt = jnp.sum(gt.astype(jnp.int32), axis=1, keepdims=True)
    eq = sk == ithr
    r = jnp.int32(k) - c_gt
    e = _cumsum_lanes(eq.astype(jnp.int32))
    sel = jnp.logical_or(gt, jnp.logical_and(eq, e <= r))
    mask = jnp.where(sel, jnp.float32(1.0 / k), jnp.float32(0.0))
    ssum = jnp.sum(mask, axis=1, keepdims=True)
    w_ref[...] = mask / (ssum + jnp.float32(1e-8))

    pos = _cumsum_lanes(sel.astype(jnp.int32)) - 1
    pos = jnp.where(sel, pos, jnp.int32(-1))
    t_iota = jax.lax.broadcasted_iota(jnp.int32, (1, T), 1)
    r_iota = jax.lax.broadcasted_iota(jnp.int32, (kpad, 1), 0)
    rows = []
    for b in range(B):
        onehot = (pos[b:b + 1] == r_iota)
        contrib = jnp.sum(jnp.where(onehot, t_iota, 0), axis=1)
        rows.append(contrib + jnp.int32(b * T))
    idx_ref[...] = jnp.stack(rows, axis=0)


def _make_sc_pool(B, T, C, k, kpad, ch):
    nch = kpad // ch
    csl = C // _LANES
    w0 = np.float32(np.float32(1.0 / k) /
                    (np.float32(k * np.float64(np.float32(1.0 / k)))
                     + np.float32(1e-8)))
    mesh = plsc.VectorSubcoreMesh(core_axis_name="c", subcore_axis_name="s")

    @functools.partial(
        pl.kernel, mesh=mesh,
        out_type=jax.ShapeDtypeStruct((B, C), jnp.float32),
        scratch_types=(
            [pltpu.VMEM((ch,), jnp.int32) for _ in range(nch)]
            + [pltpu.VMEM((kpad, C), jnp.float32),
               pltpu.VMEM((C,), jnp.float32),
               pltpu.SemaphoreType.DMA]
        ),
    )
    def sc_pool(idx_hbm, seg_hbm, clip_hbm, *refs):
        idxv = refs[:nch]
        rows, clipv, sem = refs[nch], refs[nch + 1], refs[nch + 2]
        wid = lax.axis_index("s") * 2 + lax.axis_index("c")

        @pl.when(wid < B)
        def _():
            base = wid * kpad
            for c in range(nch):
                pltpu.sync_copy(idx_hbm.at[pl.ds(base + c * ch, ch)], idxv[c])
            copies = [
                pltpu.async_copy(seg_hbm.at[idxv[c]],
                                 rows.at[pl.ds(c * ch, ch)], sem)
                for c in range(nch)
            ]
            for cp in copies:
                cp.wait()

            def acc_body(j, accs):
                return tuple(accs[s] + rows[j, pl.ds(s * _LANES, _LANES)]
                             for s in range(csl))
            accs = tuple(jnp.zeros((_LANES,), jnp.float32)
                         for _ in range(csl))
            accs = lax.fori_loop(0, k, acc_body, accs)
            for s in range(csl):
                clipv[pl.ds(s * _LANES, _LANES)] = accs[s] * w0
            pltpu.sync_copy(clipv, clip_hbm.at[wid])

    return sc_pool


def kernel(x, W1, b1, W2, b2, Wa1, ba1, Wa2, ba2):
    B, T, D = x.shape
    HID = W1.shape[0]
    C = W2.shape[0]
    k = max(1, min(T, int(round(T * _TOPK_RATIO))))
    kpad = ((k + _LANES - 1) // _LANES) * _LANES
    ch = None
    for cand in range(min(128, kpad), 7, -1):
        if cand % 8 == 0 and kpad % cand == 0:
            ch = cand
            break
    M = B * T
    TM = 2048 if M % 2048 == 0 else T

    xf = x.reshape(M, D)
    b1r = b1.reshape(1, HID)
    ba1r = ba1.reshape(1, HID)
    b2r = b2.reshape(1, C)
    ba2r = ba2.reshape(1, 1)
    wa2p = jnp.zeros((128, HID), jnp.float32).at[0].set(Wa2[0])

    seg_flat, attn_flat = pl.pallas_call(
        _fused_mm_kernel,
        grid=(M // TM,),
        in_specs=[
            pl.BlockSpec((TM, D), lambda i: (i, 0)),
            pl.BlockSpec((HID, D), lambda i: (0, 0)),
            pl.BlockSpec((1, HID), lambda i: (0, 0)),
            pl.BlockSpec((HID, D), lambda i: (0, 0)),
            pl.BlockSpec((1, HID), lambda i: (0, 0)),
            pl.BlockSpec((C, HID), lambda i: (0, 0)),
            pl.BlockSpec((1, C), lambda i: (0, 0)),
            pl.BlockSpec((128, HID), lambda i: (0, 0)),
            pl.BlockSpec((1, 1), lambda i: (0, 0)),
        ],
        out_specs=[
            pl.BlockSpec((TM, C), lambda i: (i, 0)),
            pl.BlockSpec((TM, 1), lambda i: (i, 0)),
        ],
        out_shape=[
            jax.ShapeDtypeStruct((M, C), jnp.float32),
            jax.ShapeDtypeStruct((M, 1), jnp.float32),
        ],
    )(xf, W1, b1r, Wa1, ba1r, W2, b2r, wa2p, ba2r)

    seg_logits = seg_flat.reshape(B, T, C)
    attn = attn_flat.reshape(B, T)

    weights, idx = pl.pallas_call(
        functools.partial(_weights_kernel, k=k, kpad=kpad),
        grid=(1,),
        in_specs=[pl.BlockSpec((B, T), lambda i: (0, 0))],
        out_specs=[
            pl.BlockSpec((B, T), lambda i: (0, 0)),
            pl.BlockSpec((B, kpad), lambda i: (0, 0)),
        ],
        out_shape=[
            jax.ShapeDtypeStruct((B, T), jnp.float32),
            jax.ShapeDtypeStruct((B, kpad), jnp.int32),
        ],
    )(attn)

    if ch is None:
        def _pool_kernel(w_ref, seg_ref, clip_ref):
            clip_ref[0] = jax.lax.dot_general(
                w_ref[0], seg_ref[0], (((1,), (0,)), ((), ())),
                preferred_element_type=jnp.float32)
        clip_logits = pl.pallas_call(
            _pool_kernel,
            grid=(B,),
            in_specs=[
                pl.BlockSpec((1, 1, T), lambda b: (b, 0, 0)),
                pl.BlockSpec((1, T, C), lambda b: (b, 0, 0)),
            ],
            out_specs=[pl.BlockSpec((1, 1, C), lambda b: (b, 0, 0))],
            out_shape=[jax.ShapeDtypeStruct((B, 1, C), jnp.float32)],
        )(weights.reshape(B, 1, T), seg_logits)[0].reshape(B, C)
    else:
        sc_pool = _make_sc_pool(B, T, C, k, kpad, ch)
        clip_logits = sc_pool(idx.reshape(B * kpad), seg_flat)

    return clip_logits, seg_logits, weights
